# baseline (speedup 1.0000x reference)
import functools

import jax
import jax.numpy as jnp
from jax import lax
from jax.experimental import pallas as pl
from jax.experimental.pallas import tpu as pltpu
from jax.experimental.pallas import tpu_sc as plsc

TB = 512
TK = 1024
LANES = 128
_CHUNK = 2736


def _round_bf16(v):
    r = lax.bitcast_convert_type(v, jnp.uint32)
    r = (r + jnp.uint32(0x7FFF) + ((r >> 16) & jnp.uint32(1))) & jnp.uint32(0xFFFF0000)
    return lax.bitcast_convert_type(r, jnp.float32)


def _argmin_body(cb2_ref, cbsq_ref, x16_ref, xsq_ref, idx_ref):
    K = cb2_ref.shape[0]
    x16 = x16_ref[...]
    x_sq = xsq_ref[...]
    n_tiles = K // TK
    cols_per_tile = TK // LANES

    BIGV = jnp.float32(jnp.inf)
    parts = [
        (jnp.full((TB, LANES), BIGV, jnp.float32),
         jnp.zeros((TB, LANES), jnp.int32))
        for _ in range(3)
    ]

    lane_iota = lax.broadcasted_iota(jnp.int32, (TB, LANES), 1)

    def fold_cols(pieces):
        dv, dord = pieces[0][0], jnp.full((TB, LANES), pieces[0][1], jnp.int32)
        for d, o in pieces[1:]:
            take = d < dv
            dv = jnp.where(take, d, dv)
            dord = jnp.where(take, o, dord)
        return dv, dord

    def merge(chunk, dv, dord):
        pv, po = parts[chunk]
        take = dv < pv
        parts[chunk] = (jnp.where(take, dv, pv), jnp.where(take, dord, po))

    for j in range(n_tiles):
        cb2 = cb2_ref[j * TK:(j + 1) * TK, :]
        dots2 = lax.dot_general(x16, cb2, (((1,), (1,)), ((), ())),
                                preferred_element_type=jnp.float32)
        cols = [
            (x_sq - dots2[:, c * LANES:(c + 1) * LANES])
            + cbsq_ref[j * TK + c * LANES:j * TK + (c + 1) * LANES][None, :]
            for c in range(cols_per_tile)
        ]
        ords = [j * cols_per_tile + c for c in range(cols_per_tile)]
        lo_chunk = (j * TK) // _CHUNK
        hi_chunk = ((j + 1) * TK - 1) // _CHUNK
        if lo_chunk == hi_chunk:
            dv, dord = fold_cols(list(zip(cols, ords)))
            merge(lo_chunk, dv, dord)
        else:
            split = (lo_chunk + 1) * _CHUNK - j * TK
            cfull, lrem = split // LANES, split % LANES
            lo_pieces = [(cols[c], ords[c]) for c in range(cfull)]
            hi_pieces = [(cols[c], ords[c]) for c in range(cfull + 1, cols_per_tile)]
            if lrem:
                in_lo = lane_iota < lrem
                lo_pieces.append((jnp.where(in_lo, cols[cfull], BIGV), ords[cfull]))
                hi_pieces.insert(0, (jnp.where(in_lo, BIGV, cols[cfull]), ords[cfull]))
            dv, dord = fold_cols(lo_pieces)
            merge(lo_chunk, dv, dord)
            dv, dord = fold_cols(hi_pieces)
            merge(hi_chunk, dv, dord)

    mins, idxs = [], []
    for c in range(3):
        pv, po = parts[c]
        g = po * LANES + lane_iota
        m = jnp.min(pv, axis=1, keepdims=True)
        gi = jnp.min(jnp.where(pv == m, g, K), axis=1, keepdims=True)
        mins.append(m)
        idxs.append(gi)

    v = _round_bf16(mins[0])
    i = idxs[0]
    for c in (1, 2):
        take = mins[c] < v
        v = _round_bf16(jnp.where(take, mins[c], v))
        i = jnp.where(take, idxs[c], i)
    idx_ref[...] = i[:, 0]


def _argmin_call(x16, x_sq, cb2_16, cb_sq):
    B = x16.shape[0]
    K, C = cb2_16.shape
    return pl.pallas_call(
        _argmin_body,
        grid=(B // TB,),
        in_specs=[
            pl.BlockSpec((K, C), lambda i: (0, 0)),
            pl.BlockSpec((K,), lambda i: (0,)),
            pl.BlockSpec((TB, C), lambda i: (i, 0)),
            pl.BlockSpec((TB, 1), lambda i: (i, 0)),
        ],
        out_specs=pl.BlockSpec((TB,), lambda i: (i,)),
        out_shape=jax.ShapeDtypeStruct((B,), jnp.int32),
        compiler_params=pltpu.CompilerParams(
            dimension_semantics=("parallel",),
        ),
    )(cb2_16, cb_sq, x16, x_sq)


def _gather_call(codebook, idx):
    B = idx.shape[0]
    K, C = codebook.shape
    idx2 = idx.reshape((1, B))
    mesh = plsc.VectorSubcoreMesh(core_axis_name="core",
                                  subcore_axis_name="subcore")
    W = 128

    @functools.partial(
        pl.kernel,
        out_type=jax.ShapeDtypeStruct((B, C), codebook.dtype),
        mesh=mesh,
    )
    def k(cb_hbm, i_hbm, o_hbm):
        def body(i_vmem, o_vmem):
            pltpu.sync_copy(cb_hbm.at[i_vmem.at[0]], o_vmem)

        pltpu.emit_pipeline(
            body,
            grid=(B // W,),
            in_specs=[pl.BlockSpec((1, W), index_map=lambda i: (0, i))],
            out_specs=[pl.BlockSpec((W, C), index_map=lambda i: (i, 0))],
            core_axis_name=("core", "subcore"),
            dimension_semantics=(pltpu.PARALLEL,),
        )(i_hbm, o_hbm)

    return k(codebook, idx2)


def _one_device(x, codebook):
    x_sq = jnp.sum(x * x, axis=-1, keepdims=True)
    cb_sq = jnp.sum(codebook * codebook, axis=-1)
    x16 = x.astype(jnp.bfloat16)
    cb2_16 = codebook.astype(jnp.bfloat16) * 2
    idx = _argmin_call(x16, x_sq, cb2_16, cb_sq)
    quantized = _gather_call(codebook, idx)
    return quantized, idx


def kernel(x, codebook):
    devs = [d for d in jax.devices() if d.platform == "tpu"]
    n = 2 if len(devs) >= 2 and x.shape[0] % (2 * TB) == 0 else 1
    if n == 1:
        return _one_device(x, codebook)
    mesh = jax.sharding.Mesh(devs[:n], ("d",))
    P = jax.sharding.PartitionSpec
    fn = jax.shard_map(
        _one_device, mesh=mesh,
        in_specs=(P("d", None), P(None, None)),
        out_specs=(P("d", None), P("d")),
        check_vma=False,
    )
    return fn(x, codebook)


# speedup vs baseline: 1.0124x; 1.0124x over previous
import functools

import jax
import jax.numpy as jnp
from jax import lax
from jax.experimental import pallas as pl
from jax.experimental.pallas import tpu as pltpu
from jax.experimental.pallas import tpu_sc as plsc

TB = 512
TK = 1024
LANES = 128
_CHUNK = 2736


def _round_bf16(v):
    r = lax.bitcast_convert_type(v, jnp.uint32)
    r = (r + jnp.uint32(0x7FFF) + ((r >> 16) & jnp.uint32(1))) & jnp.uint32(0xFFFF0000)
    return lax.bitcast_convert_type(r, jnp.float32)


def _argmin_body(cb2_ref, cbsq_ref, x16_ref, xsq_ref, idx_ref):
    K = cb2_ref.shape[0]
    x16 = x16_ref[...]
    x_sq = xsq_ref[...]
    n_tiles = K // TK
    cols_per_tile = TK // LANES

    BIGV = jnp.float32(jnp.inf)
    parts = [
        (jnp.full((TB, LANES), BIGV, jnp.float32),
         jnp.zeros((TB, LANES), jnp.int32))
        for _ in range(3)
    ]

    lane_iota = lax.broadcasted_iota(jnp.int32, (TB, LANES), 1)

    def fold_cols(pieces):
        dv, dord = pieces[0][0], jnp.full((TB, LANES), pieces[0][1], jnp.int32)
        for d, o in pieces[1:]:
            take = d < dv
            dv = jnp.where(take, d, dv)
            dord = jnp.where(take, o, dord)
        return dv, dord

    def merge(chunk, dv, dord):
        pv, po = parts[chunk]
        take = dv < pv
        parts[chunk] = (jnp.where(take, dv, pv), jnp.where(take, dord, po))

    for j in range(n_tiles):
        cb2 = cb2_ref[j * TK:(j + 1) * TK, :]
        dots2 = lax.dot_general(x16, cb2, (((1,), (1,)), ((), ())),
                                preferred_element_type=jnp.float32)
        cols = [
            (x_sq - dots2[:, c * LANES:(c + 1) * LANES])
            + cbsq_ref[j * TK + c * LANES:j * TK + (c + 1) * LANES][None, :]
            for c in range(cols_per_tile)
        ]
        ords = [j * cols_per_tile + c for c in range(cols_per_tile)]
        lo_chunk = (j * TK) // _CHUNK
        hi_chunk = ((j + 1) * TK - 1) // _CHUNK
        if lo_chunk == hi_chunk:
            dv, dord = fold_cols(list(zip(cols, ords)))
            merge(lo_chunk, dv, dord)
        else:
            split = (lo_chunk + 1) * _CHUNK - j * TK
            cfull, lrem = split // LANES, split % LANES
            lo_pieces = [(cols[c], ords[c]) for c in range(cfull)]
            hi_pieces = [(cols[c], ords[c]) for c in range(cfull + 1, cols_per_tile)]
            if lrem:
                in_lo = lane_iota < lrem
                lo_pieces.append((jnp.where(in_lo, cols[cfull], BIGV), ords[cfull]))
                hi_pieces.insert(0, (jnp.where(in_lo, BIGV, cols[cfull]), ords[cfull]))
            dv, dord = fold_cols(lo_pieces)
            merge(lo_chunk, dv, dord)
            dv, dord = fold_cols(hi_pieces)
            merge(hi_chunk, dv, dord)

    mins, idxs = [], []
    for c in range(3):
        pv, po = parts[c]
        g = po * LANES + lane_iota
        m = jnp.min(pv, axis=1, keepdims=True)
        gi = jnp.min(jnp.where(pv == m, g, K), axis=1, keepdims=True)
        mins.append(m)
        idxs.append(gi)

    v = _round_bf16(mins[0])
    i = idxs[0]
    for c in (1, 2):
        take = mins[c] < v
        v = _round_bf16(jnp.where(take, mins[c], v))
        i = jnp.where(take, idxs[c], i)
    idx_ref[...] = i[:, 0]


def _argmin_call(x16, x_sq, cb2_16, cb_sq):
    B = x16.shape[0]
    K, C = cb2_16.shape
    return pl.pallas_call(
        _argmin_body,
        grid=(B // TB,),
        in_specs=[
            pl.BlockSpec((K, C), lambda i: (0, 0)),
            pl.BlockSpec((K,), lambda i: (0,)),
            pl.BlockSpec((TB, C), lambda i: (i, 0)),
            pl.BlockSpec((TB, 1), lambda i: (i, 0)),
        ],
        out_specs=pl.BlockSpec((TB,), lambda i: (i,)),
        out_shape=jax.ShapeDtypeStruct((B,), jnp.int32),
        compiler_params=pltpu.CompilerParams(
            dimension_semantics=("parallel",),
        ),
    )(cb2_16, cb_sq, x16, x_sq)


def _gather_call(codebook, idx):
    B = idx.shape[0]
    K, C = codebook.shape
    idx2 = idx.reshape((1, B))
    mesh = plsc.VectorSubcoreMesh(core_axis_name="core",
                                  subcore_axis_name="subcore")
    W = 128

    @functools.partial(
        pl.kernel,
        out_type=jax.ShapeDtypeStruct((B, C), codebook.dtype),
        mesh=mesh,
    )
    def k(cb_hbm, i_hbm, o_hbm):
        def body(i_vmem, o_vmem):
            pltpu.sync_copy(cb_hbm.at[i_vmem.at[0]], o_vmem)

        pltpu.emit_pipeline(
            body,
            grid=(B // W,),
            in_specs=[pl.BlockSpec((1, W), index_map=lambda i: (0, i))],
            out_specs=[pl.BlockSpec((W, C), index_map=lambda i: (i, 0))],
            core_axis_name=("core", "subcore"),
            dimension_semantics=(pltpu.PARALLEL,),
        )(i_hbm, o_hbm)

    return k(codebook, idx2)


def _one_device(x, codebook):
    x_sq = jnp.sum(x * x, axis=-1, keepdims=True)
    cb_sq = jnp.sum(codebook * codebook, axis=-1)
    x16 = x.astype(jnp.bfloat16)
    cb2_16 = codebook.astype(jnp.bfloat16) * 2
    idx = _argmin_call(x16, x_sq, cb2_16, cb_sq)
    quantized = _gather_call(codebook, idx)
    return quantized, idx


def kernel(x, codebook):
    return _one_device(x, codebook)


# speedup vs baseline: 1.0215x; 1.0090x over previous
import functools

import jax
import jax.numpy as jnp
from jax import lax
from jax.experimental import pallas as pl
from jax.experimental.pallas import tpu as pltpu
from jax.experimental.pallas import tpu_sc as plsc

TB = 1024
TK = 1024
LANES = 128
_CHUNK = 2736


def _round_bf16(v):
    r = lax.bitcast_convert_type(v, jnp.uint32)
    r = (r + jnp.uint32(0x7FFF) + ((r >> 16) & jnp.uint32(1))) & jnp.uint32(0xFFFF0000)
    return lax.bitcast_convert_type(r, jnp.float32)


def _argmin_body(cb2_ref, cbsq_ref, x16_ref, xsq_ref, idx_ref):
    K = cb2_ref.shape[0]
    x16 = x16_ref[...]
    x_sq = xsq_ref[...]
    n_tiles = K // TK
    cols_per_tile = TK // LANES

    BIGV = jnp.float32(jnp.inf)
    parts = [
        (jnp.full((TB, LANES), BIGV, jnp.float32),
         jnp.zeros((TB, LANES), jnp.int32))
        for _ in range(3)
    ]

    lane_iota = lax.broadcasted_iota(jnp.int32, (TB, LANES), 1)

    def fold_cols(pieces):
        dv, dord = pieces[0][0], jnp.full((TB, LANES), pieces[0][1], jnp.int32)
        for d, o in pieces[1:]:
            take = d < dv
            dv = jnp.where(take, d, dv)
            dord = jnp.where(take, o, dord)
        return dv, dord

    def merge(chunk, dv, dord):
        pv, po = parts[chunk]
        take = dv < pv
        parts[chunk] = (jnp.where(take, dv, pv), jnp.where(take, dord, po))

    for j in range(n_tiles):
        cb2 = cb2_ref[j * TK:(j + 1) * TK, :]
        dots2 = lax.dot_general(x16, cb2, (((1,), (1,)), ((), ())),
                                preferred_element_type=jnp.float32)
        cols = [
            (x_sq - dots2[:, c * LANES:(c + 1) * LANES])
            + cbsq_ref[j * TK + c * LANES:j * TK + (c + 1) * LANES][None, :]
            for c in range(cols_per_tile)
        ]
        ords = [j * cols_per_tile + c for c in range(cols_per_tile)]
        lo_chunk = (j * TK) // _CHUNK
        hi_chunk = ((j + 1) * TK - 1) // _CHUNK
        if lo_chunk == hi_chunk:
            dv, dord = fold_cols(list(zip(cols, ords)))
            merge(lo_chunk, dv, dord)
        else:
            split = (lo_chunk + 1) * _CHUNK - j * TK
            cfull, lrem = split // LANES, split % LANES
            lo_pieces = [(cols[c], ords[c]) for c in range(cfull)]
            hi_pieces = [(cols[c], ords[c]) for c in range(cfull + 1, cols_per_tile)]
            if lrem:
                in_lo = lane_iota < lrem
                lo_pieces.append((jnp.where(in_lo, cols[cfull], BIGV), ords[cfull]))
                hi_pieces.insert(0, (jnp.where(in_lo, BIGV, cols[cfull]), ords[cfull]))
            dv, dord = fold_cols(lo_pieces)
            merge(lo_chunk, dv, dord)
            dv, dord = fold_cols(hi_pieces)
            merge(hi_chunk, dv, dord)

    mins, idxs = [], []
    for c in range(3):
        pv, po = parts[c]
        g = po * LANES + lane_iota
        m = jnp.min(pv, axis=1, keepdims=True)
        gi = jnp.min(jnp.where(pv == m, g, K), axis=1, keepdims=True)
        mins.append(m)
        idxs.append(gi)

    v = _round_bf16(mins[0])
    i = idxs[0]
    for c in (1, 2):
        take = mins[c] < v
        v = _round_bf16(jnp.where(take, mins[c], v))
        i = jnp.where(take, idxs[c], i)
    idx_ref[...] = i[:, 0]


def _argmin_call(x16, x_sq, cb2_16, cb_sq):
    B = x16.shape[0]
    K, C = cb2_16.shape
    return pl.pallas_call(
        _argmin_body,
        grid=(B // TB,),
        in_specs=[
            pl.BlockSpec((K, C), lambda i: (0, 0)),
            pl.BlockSpec((K,), lambda i: (0,)),
            pl.BlockSpec((TB, C), lambda i: (i, 0)),
            pl.BlockSpec((TB, 1), lambda i: (i, 0)),
        ],
        out_specs=pl.BlockSpec((TB,), lambda i: (i,)),
        out_shape=jax.ShapeDtypeStruct((B,), jnp.int32),
        compiler_params=pltpu.CompilerParams(
            dimension_semantics=("parallel",),
        ),
    )(cb2_16, cb_sq, x16, x_sq)


def _gather_call(codebook, idx):
    B = idx.shape[0]
    K, C = codebook.shape
    idx2 = idx.reshape((1, B))
    mesh = plsc.VectorSubcoreMesh(core_axis_name="core",
                                  subcore_axis_name="subcore")
    W = 128

    @functools.partial(
        pl.kernel,
        out_type=jax.ShapeDtypeStruct((B, C), codebook.dtype),
        mesh=mesh,
    )
    def k(cb_hbm, i_hbm, o_hbm):
        def body(i_vmem, o_vmem):
            pltpu.sync_copy(cb_hbm.at[i_vmem.at[0]], o_vmem)

        pltpu.emit_pipeline(
            body,
            grid=(B // W,),
            in_specs=[pl.BlockSpec((1, W), index_map=lambda i: (0, i))],
            out_specs=[pl.BlockSpec((W, C), index_map=lambda i: (i, 0))],
            core_axis_name=("core", "subcore"),
            dimension_semantics=(pltpu.PARALLEL,),
        )(i_hbm, o_hbm)

    return k(codebook, idx2)


def _one_device(x, codebook):
    x_sq = jnp.sum(x * x, axis=-1, keepdims=True)
    cb_sq = jnp.sum(codebook * codebook, axis=-1)
    x16 = x.astype(jnp.bfloat16)
    cb2_16 = codebook.astype(jnp.bfloat16) * 2
    idx = _argmin_call(x16, x_sq, cb2_16, cb_sq)
    quantized = _gather_call(codebook, idx)
    return quantized, idx


def kernel(x, codebook):
    return _one_device(x, codebook)


# speedup vs baseline: 1.0653x; 1.0429x over previous
import functools

import jax
import jax.numpy as jnp
from jax import lax
from jax.experimental import pallas as pl
from jax.experimental.pallas import tpu as pltpu
from jax.experimental.pallas import tpu_sc as plsc

TB = 2048
TK = 1024
LANES = 128
_CHUNK = 2736


def _round_bf16(v):
    r = lax.bitcast_convert_type(v, jnp.uint32)
    r = (r + jnp.uint32(0x7FFF) + ((r >> 16) & jnp.uint32(1))) & jnp.uint32(0xFFFF0000)
    return lax.bitcast_convert_type(r, jnp.float32)


def _argmin_body(cb2_ref, cbsq_ref, x16_ref, xsq_ref, idx_ref):
    K = cb2_ref.shape[0]
    x16 = x16_ref[...]
    x_sq = xsq_ref[...]
    n_tiles = K // TK
    cols_per_tile = TK // LANES

    BIGV = jnp.float32(jnp.inf)
    parts = [
        (jnp.full((TB, LANES), BIGV, jnp.float32),
         jnp.zeros((TB, LANES), jnp.int32))
        for _ in range(3)
    ]

    lane_iota = lax.broadcasted_iota(jnp.int32, (TB, LANES), 1)

    def fold_cols(pieces):
        dv, dord = pieces[0][0], jnp.full((TB, LANES), pieces[0][1], jnp.int32)
        for d, o in pieces[1:]:
            take = d < dv
            dv = jnp.where(take, d, dv)
            dord = jnp.where(take, o, dord)
        return dv, dord

    def merge(chunk, dv, dord):
        pv, po = parts[chunk]
        take = dv < pv
        parts[chunk] = (jnp.where(take, dv, pv), jnp.where(take, dord, po))

    for j in range(n_tiles):
        cb2 = cb2_ref[j * TK:(j + 1) * TK, :]
        dots2 = lax.dot_general(x16, cb2, (((1,), (1,)), ((), ())),
                                preferred_element_type=jnp.float32)
        cols = [
            (x_sq - dots2[:, c * LANES:(c + 1) * LANES])
            + cbsq_ref[j * TK + c * LANES:j * TK + (c + 1) * LANES][None, :]
            for c in range(cols_per_tile)
        ]
        ords = [j * cols_per_tile + c for c in range(cols_per_tile)]
        lo_chunk = (j * TK) // _CHUNK
        hi_chunk = ((j + 1) * TK - 1) // _CHUNK
        if lo_chunk == hi_chunk:
            dv, dord = fold_cols(list(zip(cols, ords)))
            merge(lo_chunk, dv, dord)
        else:
            split = (lo_chunk + 1) * _CHUNK - j * TK
            cfull, lrem = split // LANES, split % LANES
            lo_pieces = [(cols[c], ords[c]) for c in range(cfull)]
            hi_pieces = [(cols[c], ords[c]) for c in range(cfull + 1, cols_per_tile)]
            if lrem:
                in_lo = lane_iota < lrem
                lo_pieces.append((jnp.where(in_lo, cols[cfull], BIGV), ords[cfull]))
                hi_pieces.insert(0, (jnp.where(in_lo, BIGV, cols[cfull]), ords[cfull]))
            dv, dord = fold_cols(lo_pieces)
            merge(lo_chunk, dv, dord)
            dv, dord = fold_cols(hi_pieces)
            merge(hi_chunk, dv, dord)

    mins, idxs = [], []
    for c in range(3):
        pv, po = parts[c]
        g = po * LANES + lane_iota
        m = jnp.min(pv, axis=1, keepdims=True)
        gi = jnp.min(jnp.where(pv == m, g, K), axis=1, keepdims=True)
        mins.append(m)
        idxs.append(gi)

    v = _round_bf16(mins[0])
    i = idxs[0]
    for c in (1, 2):
        take = mins[c] < v
        v = _round_bf16(jnp.where(take, mins[c], v))
        i = jnp.where(take, idxs[c], i)
    idx_ref[...] = i[:, 0]


def _argmin_call(x16, x_sq, cb2_16, cb_sq):
    B = x16.shape[0]
    K, C = cb2_16.shape
    return pl.pallas_call(
        _argmin_body,
        grid=(B // TB,),
        in_specs=[
            pl.BlockSpec((K, C), lambda i: (0, 0)),
            pl.BlockSpec((K,), lambda i: (0,)),
            pl.BlockSpec((TB, C), lambda i: (i, 0)),
            pl.BlockSpec((TB, 1), lambda i: (i, 0)),
        ],
        out_specs=pl.BlockSpec((TB,), lambda i: (i,)),
        out_shape=jax.ShapeDtypeStruct((B,), jnp.int32),
        compiler_params=pltpu.CompilerParams(
            dimension_semantics=("parallel",),
        ),
    )(cb2_16, cb_sq, x16, x_sq)


def _gather_call(codebook, idx):
    B = idx.shape[0]
    K, C = codebook.shape
    idx2 = idx.reshape((1, B))
    mesh = plsc.VectorSubcoreMesh(core_axis_name="core",
                                  subcore_axis_name="subcore")
    W = 128

    @functools.partial(
        pl.kernel,
        out_type=jax.ShapeDtypeStruct((B, C), codebook.dtype),
        mesh=mesh,
    )
    def k(cb_hbm, i_hbm, o_hbm):
        def body(i_vmem, o_vmem):
            pltpu.sync_copy(cb_hbm.at[i_vmem.at[0]], o_vmem)

        pltpu.emit_pipeline(
            body,
            grid=(B // W,),
            in_specs=[pl.BlockSpec((1, W), index_map=lambda i: (0, i))],
            out_specs=[pl.BlockSpec((W, C), index_map=lambda i: (i, 0))],
            core_axis_name=("core", "subcore"),
            dimension_semantics=(pltpu.PARALLEL,),
        )(i_hbm, o_hbm)

    return k(codebook, idx2)


def _one_device(x, codebook):
    x_sq = jnp.sum(x * x, axis=-1, keepdims=True)
    cb_sq = jnp.sum(codebook * codebook, axis=-1)
    x16 = x.astype(jnp.bfloat16)
    cb2_16 = codebook.astype(jnp.bfloat16) * 2
    idx = _argmin_call(x16, x_sq, cb2_16, cb_sq)
    quantized = _gather_call(codebook, idx)
    return quantized, idx


def kernel(x, codebook):
    return _one_device(x, codebook)
